# baseline (speedup 1.0000x reference)
import jax
import jax.numpy as jnp
import numpy as np
from jax import lax
from jax.experimental import pallas as pl
from jax.experimental.pallas import tpu as pltpu
from jax.experimental.pallas import tpu_sc as plsc

SR = 22050
AUDIO_DURATION = 30
N_FFT = 1024
HOP = 256
INTERVAL = HOP / SR
FIXED_LEN = AUDIO_DURATION * SR // HOP + 1
N_SAMPLES = AUDIO_DURATION * SR
N_FRAMES = 1 + N_SAMPLES // HOP
N_FRAMES_PAD = 2592
NK = 512
NBINS = 88

F_BLK = 144
N_GRID = N_FRAMES_PAD // F_BLK

NTILES = 32
FPT = 81
STRIP_W = 82
STRIP_SZ = NBINS * STRIP_W
EPT = FPT * NK
NVEC = EPT // 16


def _host_consts():
    k = np.arange(N_FFT)
    win = (0.5 - 0.5 * np.cos(2.0 * np.pi * k / N_FFT)).astype(np.float32)
    wp = np.concatenate([win[-1:], win, win[:1]])
    dwin = np.gradient(wp)[1:-1].astype(np.float32)

    n = np.arange(N_FFT, dtype=np.float64)[:, None]
    kk = np.arange(NK, dtype=np.float64)[None, :]
    ang = 2.0 * np.pi * n * kk / N_FFT
    c, s = np.cos(ang), -np.sin(ang)
    m32 = np.concatenate(
        [win[:, None] * c, win[:, None] * s, dwin[:, None] * c, dwin[:, None] * s],
        axis=1,
    ).astype(np.float32)
    m_hi = m32.astype(jnp.bfloat16)
    m_lo = (m32 - m_hi.astype(np.float32)).astype(jnp.bfloat16)

    ratio = 1.059463094
    lowest = 27.5
    hz = [lowest * ratio ** i for i in range(89)]
    fb = np.array(
        [(x + y) / 2.0 for x, y in zip([lowest / ratio] + hz, hz)], dtype=np.float64
    ).astype(np.float32)

    tb = np.arange(0.0, AUDIO_DURATION, INTERVAL).astype(np.float32)
    t = np.arange(N_FRAMES).astype(np.float32) * np.float32(HOP / SR)
    nt = tb.shape[0] - 1
    ti = np.searchsorted(tb, t, side="right") - 1
    ti = np.where(t == tb[-1], nt - 1, ti)
    assert np.all((ti >= 0) & (ti <= nt - 1))

    tile_of = np.arange(N_FRAMES_PAD) // FPT
    ti_pad = np.concatenate([ti, 81 * tile_of[N_FRAMES:] - 1])
    tilocal = ti_pad - (FPT * tile_of - 1)
    assert np.all((tilocal >= 0) & (tilocal < STRIP_W))

    ub = fb.view(np.int32)
    mant = ub & np.int32(0x7FFFFF)
    mset = np.sort(
        np.array([np.bincount(mant[j::12]).argmax() for j in range(12)], dtype=np.int64)
    ).astype(np.int32)
    x3 = np.int32(ub[3] - 1)
    x82 = np.int32(ub[82])

    def cnt_fold(farr):
        farr = np.asarray(farr, np.float32)
        u = np.maximum(farr, np.float32(0.0)).view(np.int32)
        g = (u >> 23) * 12
        for mj in mset:
            g = g + ((u & np.int32(0x7FFFFF)) >= mj).astype(np.int32)
        return g - (u == x3).astype(np.int32) + (u == x82).astype(np.int32)

    g0s = cnt_fold(fb) - (np.arange(89) + 1)
    assert len(set(g0s.tolist())) == 1
    g0 = np.int32(g0s[0])
    chk = (np.concatenate([ub + d for d in range(-4, 5)])).view(np.float32)
    assert np.array_equal(
        np.clip(cnt_fold(chk) - g0, 0, 89),
        np.searchsorted(fb, chk, side="right").astype(np.int32),
    )

    return m_hi, m_lo, fb, tilocal.astype(np.int32), nt, mset, x3, x82, g0


(_M_HI, _M_LO, _FB, _TILOCAL, _NT, _MSET, _X3, _X82, _G0) = _host_consts()
_FB88 = np.float32(_FB[88])
_CORR_SCALE = np.float32(0.5 * SR / np.pi)
_FREQ_STEP = np.float32(SR / N_FFT)


def _k1_body(y0, y1, y2, y3, tloc, m_hi, m_lo, lin_ref, w_ref):
    fr = jnp.concatenate([y0[...], y1[...], y2[...], y3[...]], axis=1)
    fr = jnp.clip(fr, -1.0, 1.0)
    a_hi = fr.astype(jnp.bfloat16)
    a_lo = (fr - a_hi.astype(jnp.float32)).astype(jnp.bfloat16)
    dn = (((1,), (0,)), ((), ()))
    s = lax.dot_general(a_hi, m_hi[...], dn, preferred_element_type=jnp.float32)
    s = s + lax.dot_general(a_hi, m_lo[...], dn, preferred_element_type=jnp.float32)
    s = s + lax.dot_general(a_lo, m_hi[...], dn, preferred_element_type=jnp.float32)
    shr = s[:, 0:NK]
    shi = s[:, NK : 2 * NK]
    sdr = s[:, 2 * NK : 3 * NK]
    sdi = s[:, 3 * NK : 4 * NK]

    mag2 = shr * shr + shi * shi
    mag = jnp.sqrt(mag2)
    num = sdi * shr - sdr * shi
    corr = (-num / jnp.maximum(mag2, np.float32(1e-30))) * _CORR_SCALE
    kidx = lax.broadcasted_iota(jnp.int32, (F_BLK, NK), 1)
    fftf = kidx.astype(jnp.float32) * _FREQ_STEP
    f = jnp.where(mag2 > np.float32(0.0), fftf + corr, fftf)

    u = lax.bitcast_convert_type(jnp.maximum(f, np.float32(0.0)), jnp.int32)
    m = u & np.int32(0x7FFFFF)
    cnt = (u >> 23) * np.int32(12) - _G0
    for mj in _MSET:
        cnt = cnt + (m >= np.int32(mj)).astype(jnp.int32)
    cnt = cnt - (u == _X3).astype(jnp.int32) + (u == _X82).astype(jnp.int32)
    validf = (cnt >= 1) & ((cnt <= NBINS) | (f == _FB88))
    rowg = lax.broadcasted_iota(jnp.int32, (F_BLK, NK), 0) + pl.program_id(0) * F_BLK
    validf = validf & (rowg < N_FRAMES)
    fi = jnp.clip(cnt - 1, 0, NBINS - 1)
    wgt = jnp.where(validf, mag, np.float32(0.0))

    tl = tloc[...][:, 0:1]
    lin_ref[...] = jnp.where(validf, fi * STRIP_W + tl, kidx & 15)
    w_ref[...] = wgt


def _build_k1(interpret=False):
    grid = (N_GRID,)
    yspec = pl.BlockSpec((F_BLK, HOP), lambda i: (i, 0))
    return pl.pallas_call(
        _k1_body,
        grid=grid,
        in_specs=[
            yspec,
            yspec,
            yspec,
            yspec,
            pl.BlockSpec((F_BLK, 128), lambda i: (i, 0)),
            pl.BlockSpec((N_FFT, 4 * NK), lambda i: (0, 0)),
            pl.BlockSpec((N_FFT, 4 * NK), lambda i: (0, 0)),
        ],
        out_specs=[
            pl.BlockSpec((F_BLK, NK), lambda i: (i, 0)),
            pl.BlockSpec((F_BLK, NK), lambda i: (i, 0)),
        ],
        out_shape=[
            jax.ShapeDtypeStruct((N_FRAMES_PAD, NK), jnp.int32),
            jax.ShapeDtypeStruct((N_FRAMES_PAD, NK), jnp.float32),
        ],
        interpret=interpret,
    )


def _k2_body(lin_hbm, w_hbm, out_hbm, idx_v, wv_v, hist_v, s1, s2, s3, s4):
    c = lax.axis_index("c")
    s = lax.axis_index("s")
    wid = s * 2 + c
    e0 = wid * EPT
    h = EPT // 2

    cps = [
        pltpu.make_async_copy(
            lin_hbm.at[pl.ds(e0, h)], idx_v.at[pl.ds(0, h)], s1
        ),
        pltpu.make_async_copy(w_hbm.at[pl.ds(e0, h)], wv_v.at[pl.ds(0, h)], s2),
        pltpu.make_async_copy(
            lin_hbm.at[pl.ds(e0 + h, h)], idx_v.at[pl.ds(h, h)], s3
        ),
        pltpu.make_async_copy(
            w_hbm.at[pl.ds(e0 + h, h)], wv_v.at[pl.ds(h, h)], s4
        ),
    ]
    for cp in cps:
        cp.start()

    def zero(i, _):
        hist_v[pl.ds(i * 16, 16)] = jnp.zeros((16,), jnp.float32)
        return 0

    lax.fori_loop(0, STRIP_SZ // 16, zero, 0)

    def scat(base):
        def body(i, _):
            for j in range(8):
                b = base + i * 128 + j * 16
                ii = idx_v[pl.ds(b, 16)]
                vv = wv_v[pl.ds(b, 16)]
                plsc.addupdate_scatter(hist_v, [ii], vv)
            return 0

        lax.fori_loop(0, NVEC // 16, body, 0)

    cps[0].wait()
    cps[1].wait()
    scat(0)
    cps[2].wait()
    cps[3].wait()
    scat(h)

    pltpu.sync_copy(hist_v, out_hbm.at[wid])


def _build_k2(interpret=False):
    mesh = plsc.VectorSubcoreMesh(core_axis_name="c", subcore_axis_name="s")
    return pl.kernel(
        _k2_body,
        out_type=jax.ShapeDtypeStruct((NTILES, STRIP_SZ), jnp.float32),
        mesh=mesh,
        scratch_types=[
            pltpu.VMEM((EPT,), jnp.int32),
            pltpu.VMEM((EPT,), jnp.float32),
            pltpu.VMEM((STRIP_SZ,), jnp.float32),
            pltpu.SemaphoreType.DMA,
            pltpu.SemaphoreType.DMA,
            pltpu.SemaphoreType.DMA,
            pltpu.SemaphoreType.DMA,
        ],
        compiler_params=pltpu.CompilerParams(needs_layout_passes=False),
        interpret=interpret,
    )


def _k3_body(strips, out_ref, acc):
    acc[...] = jnp.zeros(acc.shape, jnp.float32)
    for w in range(NTILES):
        o = FPT * w
        acc[:, o : o + STRIP_W] += strips[:, w * STRIP_W : (w + 1) * STRIP_W]
    m = jnp.max(acc[:, 1 : _NT + 1])
    inv = np.float32(1.0) / jnp.maximum(m, np.float32(1e-12))
    out_ref[:, 0:_NT] = acc[:, 1 : _NT + 1] * inv
    out_ref[:, _NT : _NT + 1] = acc[:, _NT : _NT + 1] * inv


def _build_k3(interpret=False):
    return pl.pallas_call(
        _k3_body,
        in_specs=[pl.BlockSpec((NBINS, NTILES * STRIP_W), lambda: (0, 0))],
        out_specs=pl.BlockSpec((NBINS, FIXED_LEN), lambda: (0, 0)),
        out_shape=jax.ShapeDtypeStruct((NBINS, FIXED_LEN), jnp.float32),
        scratch_shapes=[pltpu.VMEM((NBINS, 2624), jnp.float32)],
        interpret=interpret,
    )


@jax.jit
def kernel(waveform):
    y = waveform
    if y.ndim > 1:
        y = jnp.mean(y, axis=0)
    nrow = N_FRAMES_PAD + 3
    ypad = jnp.pad(y, (N_FFT // 2, nrow * HOP - N_SAMPLES - N_FFT // 2))
    y2d = ypad.reshape(nrow, HOP)
    yv = [y2d[r : r + N_FRAMES_PAD] for r in range(4)]
    tloc = jnp.asarray(np.broadcast_to(_TILOCAL[:, None], (N_FRAMES_PAD, 128)).copy())

    lin, wgt = _build_k1()(
        yv[0], yv[1], yv[2], yv[3], tloc, jnp.asarray(_M_HI), jnp.asarray(_M_LO)
    )

    strips = _build_k2()(lin.reshape(-1), wgt.reshape(-1))

    strips_t = strips.reshape(NTILES, NBINS, STRIP_W).transpose(1, 0, 2).reshape(
        NBINS, NTILES * STRIP_W
    )
    return _build_k3()(strips_t)


# speedup vs baseline: 1.0494x; 1.0494x over previous
import jax
import jax.numpy as jnp
import numpy as np
from jax import lax
from jax.experimental import pallas as pl
from jax.experimental.pallas import tpu as pltpu
from jax.experimental.pallas import tpu_sc as plsc

SR = 22050
AUDIO_DURATION = 30
N_FFT = 1024
HOP = 256
INTERVAL = HOP / SR
FIXED_LEN = AUDIO_DURATION * SR // HOP + 1
N_SAMPLES = AUDIO_DURATION * SR
N_FRAMES = 1 + N_SAMPLES // HOP
N_FRAMES_PAD = 2592
NK = 512
NBINS = 88

F_BLK = 144
N_GRID = N_FRAMES_PAD // F_BLK

NTILES = 32
FPT = 81
STRIP_W = 82
STRIP_SZ = NBINS * STRIP_W
EPT = FPT * NK
NVEC = EPT // 16


def _host_consts():
    k = np.arange(N_FFT)
    win = (0.5 - 0.5 * np.cos(2.0 * np.pi * k / N_FFT)).astype(np.float32)
    wp = np.concatenate([win[-1:], win, win[:1]])
    dwin = np.gradient(wp)[1:-1].astype(np.float32)

    n = np.arange(N_FFT, dtype=np.float64)[:, None]
    kk = np.arange(NK, dtype=np.float64)[None, :]
    ang = 2.0 * np.pi * n * kk / N_FFT
    c, s = np.cos(ang), -np.sin(ang)
    m32 = np.concatenate(
        [win[:, None] * c, win[:, None] * s, dwin[:, None] * c, dwin[:, None] * s],
        axis=1,
    ).astype(np.float32)
    m_hi = m32.astype(jnp.bfloat16)
    m_lo = (m32 - m_hi.astype(np.float32)).astype(jnp.bfloat16)

    ratio = 1.059463094
    lowest = 27.5
    hz = [lowest * ratio ** i for i in range(89)]
    fb = np.array(
        [(x + y) / 2.0 for x, y in zip([lowest / ratio] + hz, hz)], dtype=np.float64
    ).astype(np.float32)

    tb = np.arange(0.0, AUDIO_DURATION, INTERVAL).astype(np.float32)
    t = np.arange(N_FRAMES).astype(np.float32) * np.float32(HOP / SR)
    nt = tb.shape[0] - 1
    ti = np.searchsorted(tb, t, side="right") - 1
    ti = np.where(t == tb[-1], nt - 1, ti)
    assert np.all((ti >= 0) & (ti <= nt - 1))

    tile_of = np.arange(N_FRAMES_PAD) // FPT
    ti_pad = np.concatenate([ti, 81 * tile_of[N_FRAMES:] - 1])
    tilocal = ti_pad - (FPT * tile_of - 1)
    assert np.all((tilocal >= 0) & (tilocal < STRIP_W))

    ub = fb.view(np.int32)
    mant = ub & np.int32(0x7FFFFF)
    mset = np.sort(
        np.array([np.bincount(mant[j::12]).argmax() for j in range(12)], dtype=np.int64)
    ).astype(np.int32)
    x3 = np.int32(ub[3] - 1)
    x82 = np.int32(ub[82])

    def cnt_fold(farr):
        farr = np.asarray(farr, np.float32)
        u = np.maximum(farr, np.float32(0.0)).view(np.int32)
        g = (u >> 23) * 12
        for mj in mset:
            g = g + ((u & np.int32(0x7FFFFF)) >= mj).astype(np.int32)
        return g - (u == x3).astype(np.int32) + (u == x82).astype(np.int32)

    g0s = cnt_fold(fb) - (np.arange(89) + 1)
    assert len(set(g0s.tolist())) == 1
    g0 = np.int32(g0s[0])
    chk = (np.concatenate([ub + d for d in range(-4, 5)])).view(np.float32)
    assert np.array_equal(
        np.clip(cnt_fold(chk) - g0, 0, 89),
        np.searchsorted(fb, chk, side="right").astype(np.int32),
    )

    return m_hi, m_lo, fb, tilocal.astype(np.int32), nt, mset, x3, x82, g0


(_M_HI, _M_LO, _FB, _TILOCAL, _NT, _MSET, _X3, _X82, _G0) = _host_consts()
_FB88 = np.float32(_FB[88])
_CORR_SCALE = np.float32(0.5 * SR / np.pi)
_FREQ_STEP = np.float32(SR / N_FFT)


def _k1_body(yhbm, tloc, m_hi, m_lo, lin_ref, w_ref, ybuf, sems):
    pid = pl.program_id(0)
    cur = lax.rem(pid, 2)
    nxt = lax.rem(pid + 1, 2)

    @pl.when(pid == 0)
    def _():
        pltpu.make_async_copy(
            yhbm.at[pl.ds(0, F_BLK + 8)], ybuf.at[0], sems.at[0]
        ).start()

    @pl.when(pid + 1 < N_GRID)
    def _():
        pltpu.make_async_copy(
            yhbm.at[pl.ds((pid + 1) * F_BLK, F_BLK + 8)], ybuf.at[nxt], sems.at[nxt]
        ).start()

    pltpu.make_async_copy(
        yhbm.at[pl.ds(pid * F_BLK, F_BLK + 8)], ybuf.at[cur], sems.at[cur]
    ).wait()

    yr = ybuf.at[cur]
    fr = jnp.concatenate(
        [yr[0:F_BLK], yr[1 : F_BLK + 1], yr[2 : F_BLK + 2], yr[3 : F_BLK + 3]],
        axis=1,
    )
    fr = jnp.clip(fr, -1.0, 1.0)
    a_hi = fr.astype(jnp.bfloat16)
    a_lo = (fr - a_hi.astype(jnp.float32)).astype(jnp.bfloat16)
    dn = (((1,), (0,)), ((), ()))
    s = lax.dot_general(a_hi, m_hi[...], dn, preferred_element_type=jnp.float32)
    s = s + lax.dot_general(a_hi, m_lo[...], dn, preferred_element_type=jnp.float32)
    s = s + lax.dot_general(a_lo, m_hi[...], dn, preferred_element_type=jnp.float32)
    shr = s[:, 0:NK]
    shi = s[:, NK : 2 * NK]
    sdr = s[:, 2 * NK : 3 * NK]
    sdi = s[:, 3 * NK : 4 * NK]

    mag2 = shr * shr + shi * shi
    mag = jnp.sqrt(mag2)
    num = sdi * shr - sdr * shi
    corr = (-num / jnp.maximum(mag2, np.float32(1e-30))) * _CORR_SCALE
    kidx = lax.broadcasted_iota(jnp.int32, (F_BLK, NK), 1)
    fftf = kidx.astype(jnp.float32) * _FREQ_STEP
    f = jnp.where(mag2 > np.float32(0.0), fftf + corr, fftf)

    u = lax.bitcast_convert_type(jnp.maximum(f, np.float32(0.0)), jnp.int32)
    m = u & np.int32(0x7FFFFF)
    cnt = (u >> 23) * np.int32(12) - _G0
    for mj in _MSET:
        cnt = cnt + (m >= np.int32(mj)).astype(jnp.int32)
    cnt = cnt - (u == _X3).astype(jnp.int32) + (u == _X82).astype(jnp.int32)
    validf = (cnt >= 1) & ((cnt <= NBINS) | (f == _FB88))
    rowg = lax.broadcasted_iota(jnp.int32, (F_BLK, NK), 0) + pl.program_id(0) * F_BLK
    validf = validf & (rowg < N_FRAMES)
    fi = jnp.clip(cnt - 1, 0, NBINS - 1)
    wgt = jnp.where(validf, mag, np.float32(0.0))

    tl = tloc[...][:, 0:1]
    lin_ref[...] = jnp.where(validf, fi * STRIP_W + tl, kidx & 15)
    w_ref[...] = wgt


def _build_k1(interpret=False):
    grid = (N_GRID,)
    return pl.pallas_call(
        _k1_body,
        grid=grid,
        in_specs=[
            pl.BlockSpec(memory_space=pl.ANY),
            pl.BlockSpec((F_BLK, 128), lambda i: (i, 0)),
            pl.BlockSpec((N_FFT, 4 * NK), lambda i: (0, 0)),
            pl.BlockSpec((N_FFT, 4 * NK), lambda i: (0, 0)),
        ],
        out_specs=[
            pl.BlockSpec((F_BLK, NK), lambda i: (i, 0)),
            pl.BlockSpec((F_BLK, NK), lambda i: (i, 0)),
        ],
        out_shape=[
            jax.ShapeDtypeStruct((N_FRAMES_PAD, NK), jnp.int32),
            jax.ShapeDtypeStruct((N_FRAMES_PAD, NK), jnp.float32),
        ],
        scratch_shapes=[
            pltpu.VMEM((2, F_BLK + 8, HOP), jnp.float32),
            pltpu.SemaphoreType.DMA((2,)),
        ],
        interpret=interpret,
    )


def _k2_body(lin_hbm, w_hbm, out_hbm, idx_v, wv_v, hist_v, s1, s2, s3, s4):
    c = lax.axis_index("c")
    s = lax.axis_index("s")
    wid = s * 2 + c
    e0 = wid * EPT
    h = EPT // 2

    cps = [
        pltpu.make_async_copy(
            lin_hbm.at[pl.ds(e0, h)], idx_v.at[pl.ds(0, h)], s1
        ),
        pltpu.make_async_copy(w_hbm.at[pl.ds(e0, h)], wv_v.at[pl.ds(0, h)], s2),
        pltpu.make_async_copy(
            lin_hbm.at[pl.ds(e0 + h, h)], idx_v.at[pl.ds(h, h)], s3
        ),
        pltpu.make_async_copy(
            w_hbm.at[pl.ds(e0 + h, h)], wv_v.at[pl.ds(h, h)], s4
        ),
    ]
    for cp in cps:
        cp.start()

    def zero(i, _):
        hist_v[pl.ds(i * 16, 16)] = jnp.zeros((16,), jnp.float32)
        return 0

    lax.fori_loop(0, STRIP_SZ // 16, zero, 0)

    def scat(base):
        def body(i, _):
            for j in range(8):
                b = base + i * 128 + j * 16
                ii = idx_v[pl.ds(b, 16)]
                vv = wv_v[pl.ds(b, 16)]
                plsc.addupdate_scatter(hist_v, [ii], vv)
            return 0

        lax.fori_loop(0, NVEC // 16, body, 0)

    cps[0].wait()
    cps[1].wait()
    scat(0)
    cps[2].wait()
    cps[3].wait()
    scat(h)

    pltpu.sync_copy(hist_v, out_hbm.at[wid])


def _build_k2(interpret=False):
    mesh = plsc.VectorSubcoreMesh(core_axis_name="c", subcore_axis_name="s")
    return pl.kernel(
        _k2_body,
        out_type=jax.ShapeDtypeStruct((NTILES, STRIP_SZ), jnp.float32),
        mesh=mesh,
        scratch_types=[
            pltpu.VMEM((EPT,), jnp.int32),
            pltpu.VMEM((EPT,), jnp.float32),
            pltpu.VMEM((STRIP_SZ,), jnp.float32),
            pltpu.SemaphoreType.DMA,
            pltpu.SemaphoreType.DMA,
            pltpu.SemaphoreType.DMA,
            pltpu.SemaphoreType.DMA,
        ],
        compiler_params=pltpu.CompilerParams(needs_layout_passes=False),
        interpret=interpret,
    )


def _k3_body(strips, out_ref, acc):
    acc[...] = jnp.zeros(acc.shape, jnp.float32)
    for w in range(NTILES):
        o = FPT * w
        acc[:, o : o + STRIP_W] += strips[:, w * STRIP_W : (w + 1) * STRIP_W]
    m = jnp.max(acc[:, 1 : _NT + 1])
    inv = np.float32(1.0) / jnp.maximum(m, np.float32(1e-12))
    out_ref[:, 0:_NT] = acc[:, 1 : _NT + 1] * inv
    out_ref[:, _NT : _NT + 1] = acc[:, _NT : _NT + 1] * inv


def _build_k3(interpret=False):
    return pl.pallas_call(
        _k3_body,
        in_specs=[pl.BlockSpec((NBINS, NTILES * STRIP_W), lambda: (0, 0))],
        out_specs=pl.BlockSpec((NBINS, FIXED_LEN), lambda: (0, 0)),
        out_shape=jax.ShapeDtypeStruct((NBINS, FIXED_LEN), jnp.float32),
        scratch_shapes=[pltpu.VMEM((NBINS, 2624), jnp.float32)],
        interpret=interpret,
    )


@jax.jit
def kernel(waveform):
    y = waveform
    if y.ndim > 1:
        y = jnp.mean(y, axis=0)
    nrow = N_FRAMES_PAD + 8
    ypad = jnp.pad(y, (N_FFT // 2, nrow * HOP - N_SAMPLES - N_FFT // 2))
    y2d = ypad.reshape(nrow, HOP)
    tloc = jnp.asarray(np.broadcast_to(_TILOCAL[:, None], (N_FRAMES_PAD, 128)).copy())

    lin, wgt = _build_k1()(y2d, tloc, jnp.asarray(_M_HI), jnp.asarray(_M_LO))

    strips = _build_k2()(lin.reshape(-1), wgt.reshape(-1))

    strips_t = strips.reshape(NTILES, NBINS, STRIP_W).transpose(1, 0, 2).reshape(
        NBINS, NTILES * STRIP_W
    )
    return _build_k3()(strips_t)


# speedup vs baseline: 1.0564x; 1.0067x over previous
import jax
import jax.numpy as jnp
import numpy as np
from jax import lax
from jax.experimental import pallas as pl
from jax.experimental.pallas import tpu as pltpu
from jax.experimental.pallas import tpu_sc as plsc

SR = 22050
AUDIO_DURATION = 30
N_FFT = 1024
HOP = 256
INTERVAL = HOP / SR
FIXED_LEN = AUDIO_DURATION * SR // HOP + 1
N_SAMPLES = AUDIO_DURATION * SR
N_FRAMES = 1 + N_SAMPLES // HOP
N_FRAMES_PAD = 2592
NK = 512
NBINS = 88

F_BLK = 144
N_GRID = N_FRAMES_PAD // F_BLK

NTILES = 32
FPT = 81
STRIP_W = 82
STRIP_SZ = NBINS * STRIP_W
EPT = FPT * NK
NVEC = EPT // 16


def _host_consts():
    k = np.arange(N_FFT)
    win = (0.5 - 0.5 * np.cos(2.0 * np.pi * k / N_FFT)).astype(np.float32)
    wp = np.concatenate([win[-1:], win, win[:1]])
    dwin = np.gradient(wp)[1:-1].astype(np.float32)

    n = np.arange(N_FFT, dtype=np.float64)[:, None]
    kk = np.arange(NK, dtype=np.float64)[None, :]
    ang = 2.0 * np.pi * n * kk / N_FFT
    c, s = np.cos(ang), -np.sin(ang)
    m32 = np.concatenate(
        [win[:, None] * c, win[:, None] * s, dwin[:, None] * c, dwin[:, None] * s],
        axis=1,
    ).astype(np.float32)
    m_hi = m32.astype(jnp.bfloat16)
    m_lo = (m32 - m_hi.astype(np.float32)).astype(jnp.bfloat16)

    ratio = 1.059463094
    lowest = 27.5
    hz = [lowest * ratio ** i for i in range(89)]
    fb = np.array(
        [(x + y) / 2.0 for x, y in zip([lowest / ratio] + hz, hz)], dtype=np.float64
    ).astype(np.float32)

    tb = np.arange(0.0, AUDIO_DURATION, INTERVAL).astype(np.float32)
    t = np.arange(N_FRAMES).astype(np.float32) * np.float32(HOP / SR)
    nt = tb.shape[0] - 1
    ti = np.searchsorted(tb, t, side="right") - 1
    ti = np.where(t == tb[-1], nt - 1, ti)
    assert np.all((ti >= 0) & (ti <= nt - 1))

    tile_of = np.arange(N_FRAMES_PAD) // FPT
    ti_pad = np.concatenate([ti, 81 * tile_of[N_FRAMES:] - 1])
    tilocal = ti_pad - (FPT * tile_of - 1)
    assert np.all((tilocal >= 0) & (tilocal < STRIP_W))

    ub = fb.view(np.int32)
    mant = ub & np.int32(0x7FFFFF)
    mset = np.sort(
        np.array([np.bincount(mant[j::12]).argmax() for j in range(12)], dtype=np.int64)
    ).astype(np.int32)
    x3 = np.int32(ub[3] - 1)
    x82 = np.int32(ub[82])

    def cnt_fold(farr):
        farr = np.asarray(farr, np.float32)
        u = np.maximum(farr, np.float32(0.0)).view(np.int32)
        g = (u >> 23) * 12
        for mj in mset:
            g = g + ((u & np.int32(0x7FFFFF)) >= mj).astype(np.int32)
        return g - (u == x3).astype(np.int32) + (u == x82).astype(np.int32)

    g0s = cnt_fold(fb) - (np.arange(89) + 1)
    assert len(set(g0s.tolist())) == 1
    g0 = np.int32(g0s[0])
    chk = (np.concatenate([ub + d for d in range(-4, 5)])).view(np.float32)
    assert np.array_equal(
        np.clip(cnt_fold(chk) - g0, 0, 89),
        np.searchsorted(fb, chk, side="right").astype(np.int32),
    )

    return m_hi, m_lo, fb, tilocal.astype(np.int32), nt, mset, x3, x82, g0


(_M_HI, _M_LO, _FB, _TILOCAL, _NT, _MSET, _X3, _X82, _G0) = _host_consts()
_FB88 = np.float32(_FB[88])
_CORR_SCALE = np.float32(0.5 * SR / np.pi)
_FREQ_STEP = np.float32(SR / N_FFT)


def _k1_body(frame_base, yhbm, tloc, m_hi, m_lo, lin_ref, w_ref, ybuf, sems):
    pid = pl.program_id(0)
    rb = frame_base
    cur = lax.rem(pid, 2)
    nxt = lax.rem(pid + 1, 2)

    @pl.when(pid == 0)
    def _():
        pltpu.make_async_copy(
            yhbm.at[pl.ds(rb, F_BLK + 8)], ybuf.at[0], sems.at[0]
        ).start()

    @pl.when(pid + 1 < pl.num_programs(0))
    def _():
        pltpu.make_async_copy(
            yhbm.at[pl.ds(rb + (pid + 1) * F_BLK, F_BLK + 8)], ybuf.at[nxt], sems.at[nxt]
        ).start()

    pltpu.make_async_copy(
        yhbm.at[pl.ds(rb + pid * F_BLK, F_BLK + 8)], ybuf.at[cur], sems.at[cur]
    ).wait()

    yr = ybuf.at[cur]
    fr = jnp.concatenate(
        [yr[0:F_BLK], yr[1 : F_BLK + 1], yr[2 : F_BLK + 2], yr[3 : F_BLK + 3]],
        axis=1,
    )
    fr = jnp.clip(fr, -1.0, 1.0)
    a_hi = fr.astype(jnp.bfloat16)
    a_lo = (fr - a_hi.astype(jnp.float32)).astype(jnp.bfloat16)
    dn = (((1,), (0,)), ((), ()))
    s = lax.dot_general(a_hi, m_hi[...], dn, preferred_element_type=jnp.float32)
    s = s + lax.dot_general(a_hi, m_lo[...], dn, preferred_element_type=jnp.float32)
    s = s + lax.dot_general(a_lo, m_hi[...], dn, preferred_element_type=jnp.float32)
    shr = s[:, 0:NK]
    shi = s[:, NK : 2 * NK]
    sdr = s[:, 2 * NK : 3 * NK]
    sdi = s[:, 3 * NK : 4 * NK]

    mag2 = shr * shr + shi * shi
    mag = jnp.sqrt(mag2)
    num = sdi * shr - sdr * shi
    corr = (-num / jnp.maximum(mag2, np.float32(1e-30))) * _CORR_SCALE
    kidx = lax.broadcasted_iota(jnp.int32, (F_BLK, NK), 1)
    fftf = kidx.astype(jnp.float32) * _FREQ_STEP
    f = jnp.where(mag2 > np.float32(0.0), fftf + corr, fftf)

    u = lax.bitcast_convert_type(jnp.maximum(f, np.float32(0.0)), jnp.int32)
    m = u & np.int32(0x7FFFFF)
    cnt = (u >> 23) * np.int32(12) - _G0
    for mj in _MSET:
        cnt = cnt + (m >= np.int32(mj)).astype(jnp.int32)
    cnt = cnt - (u == _X3).astype(jnp.int32) + (u == _X82).astype(jnp.int32)
    validf = (cnt >= 1) & ((cnt <= NBINS) | (f == _FB88))
    rowg = lax.broadcasted_iota(jnp.int32, (F_BLK, NK), 0) + pl.program_id(0) * F_BLK + frame_base
    validf = validf & (rowg < N_FRAMES)
    fi = jnp.clip(cnt - 1, 0, NBINS - 1)
    wgt = jnp.where(validf, mag, np.float32(0.0))

    tl = tloc[...][:, 0:1]
    lin_ref[...] = jnp.where(validf, fi * STRIP_W + tl, kidx & 15)
    w_ref[...] = wgt


N_HALF = N_FRAMES_PAD // 2


def _build_k1(frame_base, interpret=False):
    import functools
    gb = frame_base // F_BLK
    return pl.pallas_call(
        functools.partial(_k1_body, frame_base),
        grid=(N_HALF // F_BLK,),
        in_specs=[
            pl.BlockSpec(memory_space=pl.ANY),
            pl.BlockSpec((F_BLK, 128), lambda i: (i + gb, 0)),
            pl.BlockSpec((N_FFT, 4 * NK), lambda i: (0, 0)),
            pl.BlockSpec((N_FFT, 4 * NK), lambda i: (0, 0)),
        ],
        out_specs=[
            pl.BlockSpec((F_BLK, NK), lambda i: (i, 0)),
            pl.BlockSpec((F_BLK, NK), lambda i: (i, 0)),
        ],
        out_shape=[
            jax.ShapeDtypeStruct((N_HALF, NK), jnp.int32),
            jax.ShapeDtypeStruct((N_HALF, NK), jnp.float32),
        ],
        scratch_shapes=[
            pltpu.VMEM((2, F_BLK + 8, HOP), jnp.float32),
            pltpu.SemaphoreType.DMA((2,)),
        ],
        interpret=interpret,
    )


HEPT = EPT // 2


def _k2_body(lin_hbm, w_hbm, out_hbm, idx_v, wv_v, hist_v, s1, s2):
    c = lax.axis_index("c")
    s = lax.axis_index("s")
    wid = s * 2 + c
    e0 = (wid >> 1) * EPT + (wid & 1) * HEPT

    cp1 = pltpu.make_async_copy(lin_hbm.at[pl.ds(e0, HEPT)], idx_v, s1)
    cp2 = pltpu.make_async_copy(w_hbm.at[pl.ds(e0, HEPT)], wv_v, s2)
    cp1.start()
    cp2.start()

    def zero(i, _):
        hist_v[pl.ds(i * 16, 16)] = jnp.zeros((16,), jnp.float32)
        return 0

    lax.fori_loop(0, STRIP_SZ // 16, zero, 0)

    cp1.wait()
    cp2.wait()

    def body(i, _):
        for j in range(8):
            b = i * 128 + j * 16
            ii = idx_v[pl.ds(b, 16)]
            vv = wv_v[pl.ds(b, 16)]
            plsc.addupdate_scatter(hist_v, [ii], vv)
        return 0

    lax.fori_loop(0, HEPT // 128, body, 0)

    pltpu.sync_copy(hist_v, out_hbm.at[wid])


def _build_k2(interpret=False):
    mesh = plsc.VectorSubcoreMesh(core_axis_name="c", subcore_axis_name="s")
    return pl.kernel(
        _k2_body,
        out_type=jax.ShapeDtypeStruct((NTILES, STRIP_SZ), jnp.float32),
        mesh=mesh,
        scratch_types=[
            pltpu.VMEM((HEPT,), jnp.int32),
            pltpu.VMEM((HEPT,), jnp.float32),
            pltpu.VMEM((STRIP_SZ,), jnp.float32),
            pltpu.SemaphoreType.DMA,
            pltpu.SemaphoreType.DMA,
        ],
        compiler_params=pltpu.CompilerParams(needs_layout_passes=False),
        interpret=interpret,
    )


def _k3_body(sa, sb, out_ref, acc):
    acc[...] = jnp.zeros(acc.shape, jnp.float32)
    for t in range(NTILES):
        o = FPT * (t >> 1)
        acc[:, o : o + STRIP_W] += sa[:, t * STRIP_W : (t + 1) * STRIP_W]
    for t in range(NTILES):
        o = FPT * (16 + (t >> 1))
        acc[:, o : o + STRIP_W] += sb[:, t * STRIP_W : (t + 1) * STRIP_W]
    m = jnp.max(acc[:, 1 : _NT + 1])
    inv = np.float32(1.0) / jnp.maximum(m, np.float32(1e-12))
    out_ref[:, 0:_NT] = acc[:, 1 : _NT + 1] * inv
    out_ref[:, _NT : _NT + 1] = acc[:, _NT : _NT + 1] * inv


def _build_k3(interpret=False):
    return pl.pallas_call(
        _k3_body,
        in_specs=[
            pl.BlockSpec((NBINS, NTILES * STRIP_W), lambda: (0, 0)),
            pl.BlockSpec((NBINS, NTILES * STRIP_W), lambda: (0, 0)),
        ],
        out_specs=pl.BlockSpec((NBINS, FIXED_LEN), lambda: (0, 0)),
        out_shape=jax.ShapeDtypeStruct((NBINS, FIXED_LEN), jnp.float32),
        scratch_shapes=[pltpu.VMEM((NBINS, 2624), jnp.float32)],
        interpret=interpret,
    )


@jax.jit
def kernel(waveform):
    y = waveform
    if y.ndim > 1:
        y = jnp.mean(y, axis=0)
    nrow = N_FRAMES_PAD + 8
    ypad = jnp.pad(y, (N_FFT // 2, nrow * HOP - N_SAMPLES - N_FFT // 2))
    y2d = ypad.reshape(nrow, HOP)
    tloc = jnp.asarray(np.broadcast_to(_TILOCAL[:, None], (N_FRAMES_PAD, 128)).copy())

    mh, ml = jnp.asarray(_M_HI), jnp.asarray(_M_LO)

    def tsp(st):
        return st.reshape(NTILES, NBINS, STRIP_W).transpose(1, 0, 2).reshape(
            NBINS, NTILES * STRIP_W
        )

    lin_a, w_a = _build_k1(0)(y2d, tloc, mh, ml)
    strips_a = _build_k2()(lin_a.reshape(-1), w_a.reshape(-1))
    lin_b, w_b = _build_k1(N_HALF)(y2d, tloc, mh, ml)
    strips_b = _build_k2()(lin_b.reshape(-1), w_b.reshape(-1))
    return _build_k3()(tsp(strips_a), tsp(strips_b))
